# baseline (speedup 1.0000x reference)
import functools

import jax
import jax.numpy as jnp
from jax import lax
from jax.experimental import pallas as pl
from jax.experimental.pallas import tpu as pltpu
from jax.experimental.pallas import tpu_sc as plsc

NC = 2
NS = 16
NW = NC * NS
LANES = 16
R_TC = 512
W_SC = R_TC // 2

NEG = float("-inf")


def _sc_stage(flat1d, starts, ends, offs, d, b):
    mesh = plsc.VectorSubcoreMesh(
        core_axis_name="c", subcore_axis_name="s", num_cores=NC, num_subcores=NS
    )

    @functools.partial(
        pl.kernel,
        out_type=jax.ShapeDtypeStruct((NW * b * d,), jnp.float32),
        mesh=mesh,
        scratch_types=[
            pltpu.VMEM((W_SC * d,), jnp.float32),
            pltpu.VMEM((b,), jnp.int32),
            pltpu.VMEM((b,), jnp.int32),
            pltpu.VMEM((LANES,), jnp.int32),
            pltpu.VMEM((b * d,), jnp.float32),
            pltpu.SemaphoreType.DMA,
        ],
    )
    def k(flat_hbm, st_hbm, en_hbm, off_hbm, out_hbm, buf, st_v, en_v, off_v, acc_v, sem):
        cid = lax.axis_index("c")
        sid = lax.axis_index("s")
        wid = sid * NC + cid

        pltpu.sync_copy(off_hbm.at[pl.ds(wid * LANES, LANES)], off_v)
        base = pl.multiple_of(off_v[...][0], W_SC * d)
        pltpu.make_async_copy(flat_hbm.at[pl.ds(base, W_SC * d)], buf, sem).start()

        pltpu.sync_copy(st_hbm.at[pl.ds(wid * b, b)], st_v)
        pltpu.sync_copy(en_hbm.at[pl.ds(wid * b, b)], en_v)
        st_vec = st_v[...]
        en_vec = en_v[...]

        neg = jnp.full((LANES,), NEG, jnp.float32)
        for kk in range(b * d // LANES):
            acc_v[pl.ds(kk * LANES, LANES)] = neg

        pltpu.make_async_copy(flat_hbm.at[pl.ds(base, W_SC * d)], buf, sem).wait()
        for s in range(b):
            lo = st_vec[s]
            hi = en_vec[s]
            accs = tuple(
                acc_v[pl.ds(s * d + LANES * j, LANES)] for j in range(d // LANES)
            )

            def rbody(r, a):
                off = r * d
                return tuple(
                    jnp.maximum(aj, buf[pl.ds(off + LANES * j, LANES)])
                    for j, aj in enumerate(a)
                )

            accs = plsc.parallel_loop(lo, hi, unroll=4, carry=accs)(rbody)
            for j in range(d // LANES):
                acc_v[pl.ds(s * d + LANES * j, LANES)] = accs[j]

        pltpu.sync_copy(acc_v, out_hbm.at[pl.ds(wid * b * d, b * d)])

    return k(flat1d, starts, ends, offs)


def _tc_blocks(flat, segmap, n, d, b):
    nblk = n // R_TC

    def body(seg_ref, x_ref, o_ref):
        i = pl.program_id(0)

        @pl.when(i == 0)
        def _():
            o_ref[...] = jnp.full((b, d), NEG, jnp.float32)

        seg = seg_ref[i]
        bm = jnp.max(x_ref[...], axis=0, keepdims=True)
        segid = lax.broadcasted_iota(jnp.int32, (b, 1), 0)
        upd = jnp.maximum(o_ref[...], bm)
        o_ref[...] = jnp.where(segid == seg, upd, o_ref[...])

    return pl.pallas_call(
        body,
        grid=(nblk,),
        in_specs=[
            pl.BlockSpec(memory_space=pltpu.SMEM),
            pl.BlockSpec((R_TC, d), lambda i: (i, 0)),
        ],
        out_specs=pl.BlockSpec((b, d), lambda i: (0, 0)),
        out_shape=jax.ShapeDtypeStruct((b, d), jnp.float32),
    )(segmap, flat)


def _tc_merge(partials_sc, partial_tc, b, d):
    def body(p_ref, q_ref, o_ref):
        acc = q_ref[...]
        for w in range(NW):
            acc = jnp.maximum(acc, p_ref[w * b : (w + 1) * b, :])
        o_ref[...] = acc

    return pl.pallas_call(
        body,
        out_shape=jax.ShapeDtypeStruct((b, d), jnp.float32),
    )(partials_sc, partial_tc)


def kernel(flat, cu_seqlens):
    n, d = flat.shape
    b = cu_seqlens.shape[0] - 1
    nblk = n // R_TC
    assert n % R_TC == 0 and d % LANES == 0

    cu = cu_seqlens.astype(jnp.int32)

    r0 = jnp.arange(nblk, dtype=jnp.int32) * R_TC
    inner = cu[1:b][None, :]
    s_first = jnp.sum(inner <= r0[:, None], axis=1, dtype=jnp.int32)
    s_last = jnp.sum(inner <= (r0 + R_TC - 1)[:, None], axis=1, dtype=jnp.int32)
    segmap = jnp.where(s_first == s_last, s_first, jnp.int32(b))

    w = jnp.arange(NW, dtype=jnp.int32)
    t = jnp.minimum(w // 2 + 1, b - 1)
    win = (cu[t] // R_TC) * R_TC
    wbase = win + (w % 2) * W_SC
    starts = jnp.clip(cu[None, :-1], wbase[:, None], wbase[:, None] + W_SC)
    ends = jnp.clip(cu[None, 1:], wbase[:, None], wbase[:, None] + W_SC)
    starts = (starts - wbase[:, None]).reshape(-1)
    ends = (ends - wbase[:, None]).reshape(-1)
    offs = jnp.repeat(wbase * d, LANES)

    partials_sc = _sc_stage(flat.reshape(-1), starts, ends, offs, d, b)
    partial_tc = _tc_blocks(flat, segmap, n, d, b)
    return _tc_merge(partials_sc.reshape(NW * b, d), partial_tc, b, d)
